# speedup vs baseline: 5.2907x; 5.2907x over previous
import functools

import jax
import jax.numpy as jnp
from jax import lax
from jax.experimental import pallas as pl
from jax.experimental.pallas import tpu as pltpu
from jax.experimental.pallas import tpu_sc as plsc

DIM = 128
EPS = 1e-12
NUM_WORKERS = 32
CHUNK = 128


def _sc_token_gather(x_flat, token_table):
    n = x_flat.shape[0]
    per_w = n // NUM_WORKERS
    n_chunks = per_w // CHUNK
    mesh = plsc.VectorSubcoreMesh(core_axis_name="c", subcore_axis_name="s")

    @functools.partial(
        pl.kernel,
        mesh=mesh,
        out_type=jax.ShapeDtypeStruct((n, DIM), jnp.float32),
        scratch_types=[
            pltpu.VMEM((CHUNK,), jnp.int32),
            pltpu.VMEM((CHUNK, DIM), jnp.float32),
            pltpu.SemaphoreType.DMA,
        ],
    )
    def k(x_ref, tab_ref, out_ref, idx_v, rows_v, sem):
        num_cores = 2
        wid = lax.axis_index("s") * num_cores + lax.axis_index("c")
        base_w = wid * per_w

        def body(c, carry):
            base = base_w + c * CHUNK
            pltpu.sync_copy(x_ref.at[pl.ds(base, CHUNK)], idx_v)
            pltpu.async_copy(tab_ref.at[idx_v], rows_v, sem).wait()
            pltpu.sync_copy(rows_v, out_ref.at[pl.ds(base, CHUNK)])
            return carry

        lax.fori_loop(0, n_chunks, body, 0)

    return k(x_flat, token_table)


def _tc_ln(h, ttf, pos, type_table, gamma, beta, seq_len):
    n = h.shape[0]
    rows = 16 * seq_len
    grid = (n // rows,)

    def body(h_ref, tt_ref, pos_ref, type_ref, g_ref, b_ref, o_ref):
        x = h_ref[...]
        x = (x.reshape(rows // seq_len, seq_len, DIM) + pos_ref[...][None]
             ).reshape(rows, DIM)
        t0 = type_ref[0:1, :]
        dt = type_ref[1:2, :] - t0
        x = x + t0 + tt_ref[...] * dt
        mean = jnp.mean(x, axis=-1, keepdims=True)
        xc = x - mean
        var = jnp.mean(xc * xc, axis=-1, keepdims=True)
        o_ref[...] = xc * lax.rsqrt(var + EPS) * g_ref[...] + b_ref[...]

    return pl.pallas_call(
        body,
        grid=grid,
        in_specs=[
            pl.BlockSpec((rows, DIM), lambda i: (i, 0)),
            pl.BlockSpec((rows, 1), lambda i: (i, 0)),
            pl.BlockSpec((seq_len, DIM), lambda i: (0, 0)),
            pl.BlockSpec((2, DIM), lambda i: (0, 0)),
            pl.BlockSpec((1, DIM), lambda i: (0, 0)),
            pl.BlockSpec((1, DIM), lambda i: (0, 0)),
        ],
        out_specs=pl.BlockSpec((rows, DIM), lambda i: (i, 0)),
        out_shape=jax.ShapeDtypeStruct((n, DIM), jnp.float32),
    )(h, ttf, pos, type_table, gamma, beta)


def kernel(x, token_type, token_table, pos_table, type_table, gamma, beta):
    b, l = x.shape
    n = b * l
    x_flat = x.reshape(n).astype(jnp.int32)
    h = _sc_token_gather(x_flat, token_table)
    ttf = token_type.reshape(n, 1).astype(jnp.float32)
    out = _tc_ln(h, ttf, pos_table[:l], type_table,
                 gamma.reshape(1, DIM), beta.reshape(1, DIM), l)
    return out.reshape(b, l, DIM)


# speedup vs baseline: 6.0611x; 1.1456x over previous
import functools

import jax
import jax.numpy as jnp
from jax import lax
from jax.experimental import pallas as pl
from jax.experimental.pallas import tpu as pltpu
from jax.experimental.pallas import tpu_sc as plsc

DIM = 128
EPS = 1e-12
NUM_WORKERS = 32
CHUNK = 128


def _sc_token_gather(x2d, token_table):
    n = x2d.shape[0] * x2d.shape[1] * CHUNK
    per_w = n // NUM_WORKERS
    n_chunks = per_w // CHUNK
    mesh = plsc.VectorSubcoreMesh(core_axis_name="c", subcore_axis_name="s")

    @functools.partial(
        pl.kernel,
        mesh=mesh,
        out_type=jax.ShapeDtypeStruct((n, DIM), jnp.float32),
        scratch_types=[
            pltpu.VMEM((n_chunks, CHUNK), jnp.int32),
            pltpu.VMEM((CHUNK, DIM), jnp.float32),
            pltpu.VMEM((CHUNK, DIM), jnp.float32),
            pltpu.SemaphoreType.DMA,
            pltpu.SemaphoreType.DMA,
            pltpu.SemaphoreType.DMA,
            pltpu.SemaphoreType.DMA,
        ],
    )
    def k(x_ref, tab_ref, out_ref, idx_all, r0, r1, gsem0, gsem1, wsem0,
          wsem1):
        num_cores = 2
        wid = lax.axis_index("s") * num_cores + lax.axis_index("c")
        base_w = wid * per_w

        def g_start(c, buf, sem):
            pltpu.async_copy(tab_ref.at[idx_all.at[c]], buf, sem)

        def g_wait(c, buf, sem):
            pltpu.make_async_copy(tab_ref.at[idx_all.at[c]], buf, sem).wait()

        def w_start(c, buf, sem):
            pltpu.async_copy(buf, out_ref.at[pl.ds(base_w + c * CHUNK, CHUNK)],
                             sem)

        def w_wait(c, buf, sem):
            pltpu.make_async_copy(
                buf, out_ref.at[pl.ds(base_w + c * CHUNK, CHUNK)], sem).wait()

        pltpu.sync_copy(x_ref.at[wid], idx_all)
        g_start(0, r0, gsem0)
        g_start(1, r1, gsem1)
        g_wait(0, r0, gsem0)
        w_start(0, r0, wsem0)

        def body(g, carry):
            c = 2 * g
            w_wait(c - 2, r0, wsem0)
            g_start(c, r0, gsem0)
            g_wait(c - 1, r1, gsem1)
            w_start(c - 1, r1, wsem1)
            w_wait(c - 1, r1, wsem1)
            g_start(c + 1, r1, gsem1)
            g_wait(c, r0, gsem0)
            w_start(c, r0, wsem0)
            return carry

        lax.fori_loop(1, n_chunks // 2, body, 0)
        g_wait(n_chunks - 1, r1, gsem1)
        w_start(n_chunks - 1, r1, wsem1)
        w_wait(n_chunks - 2, r0, wsem0)
        w_wait(n_chunks - 1, r1, wsem1)

    return k(x2d, token_table)


def _tc_ln(h, ttf, pos, type_table, gamma, beta, seq_len):
    n = h.shape[0]
    rows = 16 * seq_len
    grid = (n // rows,)

    def body(h_ref, tt_ref, pos_ref, type_ref, g_ref, b_ref, o_ref):
        x = h_ref[...]
        x = (x.reshape(rows // seq_len, seq_len, DIM) + pos_ref[...][None]
             ).reshape(rows, DIM)
        t0 = type_ref[0:1, :]
        dt = type_ref[1:2, :] - t0
        x = x + t0 + tt_ref[...] * dt
        mean = jnp.mean(x, axis=-1, keepdims=True)
        xc = x - mean
        var = jnp.mean(xc * xc, axis=-1, keepdims=True)
        o_ref[...] = xc * lax.rsqrt(var + EPS) * g_ref[...] + b_ref[...]

    return pl.pallas_call(
        body,
        grid=grid,
        in_specs=[
            pl.BlockSpec((rows, DIM), lambda i: (i, 0)),
            pl.BlockSpec((rows, 1), lambda i: (i, 0)),
            pl.BlockSpec((seq_len, DIM), lambda i: (0, 0)),
            pl.BlockSpec((2, DIM), lambda i: (0, 0)),
            pl.BlockSpec((1, DIM), lambda i: (0, 0)),
            pl.BlockSpec((1, DIM), lambda i: (0, 0)),
        ],
        out_specs=pl.BlockSpec((rows, DIM), lambda i: (i, 0)),
        out_shape=jax.ShapeDtypeStruct((n, DIM), jnp.float32),
    )(h, ttf, pos, type_table, gamma, beta)


def kernel(x, token_type, token_table, pos_table, type_table, gamma, beta):
    b, l = x.shape
    n = b * l
    x3d = x.reshape(NUM_WORKERS, n // (NUM_WORKERS * CHUNK),
                    CHUNK).astype(jnp.int32)
    h = _sc_token_gather(x3d, token_table)
    ttf = token_type.reshape(n, 1).astype(jnp.float32)
    out = _tc_ln(h, ttf, pos_table[:l], type_table,
                 gamma.reshape(1, DIM), beta.reshape(1, DIM), l)
    return out.reshape(b, l, DIM)
